# baseline (speedup 1.0000x reference)
import jax
import jax.numpy as jnp
from jax.experimental import pallas as pl

B = 256
N_MOL = 10000
N_PPI = 2000
N_PRO = 20000


def _gcn(x, edge_index, W, b, n):
    h = x @ W
    loop = jnp.arange(n)
    src = jnp.concatenate([edge_index[0], loop])
    dst = jnp.concatenate([edge_index[1], loop])
    deg = jax.ops.segment_sum(jnp.ones(src.shape[0], jnp.float32), dst, num_segments=n)
    dinv = jnp.where(deg > 0, jax.lax.rsqrt(deg), 0.0)
    norm = (dinv[src] * dinv[dst])[:, None]
    return jax.ops.segment_sum(h[src] * norm, dst, num_segments=n) + b


def _gep(x, batch, num_seg):
    s = jax.ops.segment_sum(x, batch, num_segments=num_seg)
    c = jax.ops.segment_sum(jnp.ones((x.shape[0], 1), jnp.float32), batch, num_segments=num_seg)
    return s / jnp.maximum(c, 1.0)


def _head_kernel(x_ref, w1_ref, b1_ref, w2_ref, b2_ref, wo_ref, bo_ref, o_ref):
    x = x_ref[...]
    h1 = jnp.maximum(jnp.dot(x, w1_ref[...], preferred_element_type=jnp.float32) + b1_ref[...], 0.0)
    h2 = jnp.maximum(jnp.dot(h1, w2_ref[...], preferred_element_type=jnp.float32) + b2_ref[...], 0.0)
    o_ref[...] = jnp.dot(h2, wo_ref[...], preferred_element_type=jnp.float32) + bo_ref[...]


def _head(xc, p):
    w1, b1 = p['fc1']
    w2, b2 = p['fc2']
    wo, bo = p['out']
    return pl.pallas_call(
        _head_kernel,
        out_shape=jax.ShapeDtypeStruct((B, 1), jnp.float32),
    )(xc, w1, b1[None, :], w2, b2[None, :], wo, bo[None, :])


def kernel(mol_x, mol_edge_index, mol_batch, seq_num, ppi_edge, ppi_features, pro_x, pro_edge_index, pro_graph_num, pro_batch, params):
    p = params
    relu = jax.nn.relu
    x = relu(_gcn(mol_x, mol_edge_index, p['mg1'][0], p['mg1'][1], N_MOL))
    x = relu(_gcn(x, mol_edge_index, p['mg2'][0], p['mg2'][1], N_MOL))
    x = relu(_gcn(x, mol_edge_index, p['mg3'][0], p['mg3'][1], N_MOL))
    x = _gep(x, mol_batch, B)
    x = relu(x @ p['mfc1'][0] + p['mfc1'][1])
    x = x @ p['mfc2'][0] + p['mfc2'][1]
    ppi_x = relu(_gcn(ppi_features, ppi_edge, p['ppig1'][0], p['ppig1'][1], N_PPI))
    ppi_x = relu(_gcn(ppi_x, ppi_edge, p['ppig2'][0], p['ppig2'][1], N_PPI))
    ppi_x = relu(ppi_x @ p['ppifc1'][0] + p['ppifc1'][1])
    ppi_x = ppi_x @ p['ppifc2'][0] + p['ppifc2'][1]
    p_x = relu(_gcn(pro_x, pro_edge_index, p['pg1'][0], p['pg1'][1], N_PRO))
    ppi_xx = ppi_x[pro_graph_num][pro_batch]
    p_x = jnp.concatenate([p_x + ppi_xx, p_x - ppi_xx], axis=-1)
    p_x = relu(_gcn(p_x, pro_edge_index, p['pg2'][0], p['pg2'][1], N_PRO))
    p_x = relu(_gcn(p_x, pro_edge_index, p['pg3'][0], p['pg3'][1], N_PRO))
    p_x = _gep(p_x, pro_batch, B)
    p_x = relu(p_x @ p['pfc1'][0] + p['pfc1'][1])
    p_x = p_x @ p['pfc2'][0] + p['pfc2'][1]
    p_x = p_x[seq_num]
    xc = jnp.concatenate([x, p_x], axis=1)
    return _head(xc, p)


# speedup vs baseline: 2.3557x; 2.3557x over previous
import jax
import jax.numpy as jnp
from jax import lax
from jax.experimental import pallas as pl

B = 256
N_MOL, E_MOL = 10000, 160000
N_PPI, E_PPI = 2000, 64000
N_PRO, E_PRO = 20000, 320000

f32 = jnp.float32
i32 = jnp.int32


def _mm(x, w, b=None, relu=False, rowscale=None, mb=2000):
    m, k = x.shape
    do = w.shape[1]
    mb = min(mb, m)

    def kern(*refs):
        ri = 2
        y = jnp.dot(refs[0][...], refs[1][...], preferred_element_type=f32)
        if b is not None:
            y = y + refs[ri][...]; ri += 1
        if relu:
            y = jnp.maximum(y, 0.0)
        if rowscale is not None:
            y = y * refs[ri][...]; ri += 1
        refs[-1][...] = y

    in_specs = [pl.BlockSpec((mb, k), lambda i: (i, 0)),
                pl.BlockSpec((k, do), lambda i: (0, 0))]
    ins = [x, w]
    if b is not None:
        in_specs.append(pl.BlockSpec((1, do), lambda i: (0, 0)))
        ins.append(b.reshape(1, do))
    if rowscale is not None:
        in_specs.append(pl.BlockSpec((mb, 1), lambda i: (i, 0)))
        ins.append(rowscale)
    return pl.pallas_call(
        kern, grid=(m // mb,), in_specs=in_specs,
        out_specs=pl.BlockSpec((mb, do), lambda i: (i, 0)),
        out_shape=jax.ShapeDtypeStruct((m, do), f32))(*ins)


def _combine_mm(s, g, dinv, w, bpre=None, relu_pre=False, bpost=None,
                relu_post=False, rowscale=None, mb=2000):
    m, d = g.shape
    do = w.shape[1]
    mb = min(mb, m)

    def kern(*refs):
        s_r, g_r, d_r, w_r = refs[0], refs[1], refs[2], refs[3]
        ri = 4
        t = (s_r[...] + g_r[...]) * d_r[...]
        if bpre is not None:
            t = t + refs[ri][...]; ri += 1
        if relu_pre:
            t = jnp.maximum(t, 0.0)
        y = jnp.dot(t, w_r[...], preferred_element_type=f32)
        if bpost is not None:
            y = y + refs[ri][...]; ri += 1
        if relu_post:
            y = jnp.maximum(y, 0.0)
        if rowscale is not None:
            y = y * refs[ri][...]; ri += 1
        refs[-1][...] = y

    in_specs = [pl.BlockSpec((mb, d), lambda i: (i, 0)),
                pl.BlockSpec((mb, d), lambda i: (i, 0)),
                pl.BlockSpec((mb, 1), lambda i: (i, 0)),
                pl.BlockSpec((d, do), lambda i: (0, 0))]
    ins = [s, g, dinv, w]
    if bpre is not None:
        in_specs.append(pl.BlockSpec((1, d), lambda i: (0, 0)))
        ins.append(bpre.reshape(1, d))
    if bpost is not None:
        in_specs.append(pl.BlockSpec((1, do), lambda i: (0, 0)))
        ins.append(bpost.reshape(1, do))
    if rowscale is not None:
        in_specs.append(pl.BlockSpec((mb, 1), lambda i: (i, 0)))
        ins.append(rowscale)
    return pl.pallas_call(
        kern, grid=(m // mb,), in_specs=in_specs,
        out_specs=pl.BlockSpec((mb, do), lambda i: (i, 0)),
        out_shape=jax.ShapeDtypeStruct((m, do), f32))(*ins)


def _combine(s, g, dinv, bpre=None, relu=False, mb=2000):
    m, d = g.shape
    mb = min(mb, m)

    def kern(*refs):
        t = (refs[0][...] + refs[1][...]) * refs[2][...]
        if bpre is not None:
            t = t + refs[3][...]
        if relu:
            t = jnp.maximum(t, 0.0)
        refs[-1][...] = t

    in_specs = [pl.BlockSpec((mb, d), lambda i: (i, 0)),
                pl.BlockSpec((mb, d), lambda i: (i, 0)),
                pl.BlockSpec((mb, 1), lambda i: (i, 0))]
    ins = [s, g, dinv]
    if bpre is not None:
        in_specs.append(pl.BlockSpec((1, d), lambda i: (0, 0)))
        ins.append(bpre.reshape(1, d))
    return pl.pallas_call(
        kern, grid=(m // mb,), in_specs=in_specs,
        out_specs=pl.BlockSpec((mb, d), lambda i: (i, 0)),
        out_shape=jax.ShapeDtypeStruct((m, d), f32))(*ins)


def _dinv(deg):
    n = deg.shape[0]

    def kern(d_r, o_r):
        o_r[...] = lax.rsqrt(d_r[...] + 1.0)

    return pl.pallas_call(
        kern, out_shape=jax.ShapeDtypeStruct((n, 1), f32))(deg.reshape(n, 1))


def _scale(x, dinv, mb=2000):
    m, d = x.shape

    def kern(x_r, d_r, o_r):
        o_r[...] = x_r[...] * d_r[...]

    return pl.pallas_call(
        kern, grid=(m // mb,),
        in_specs=[pl.BlockSpec((mb, d), lambda i: (i, 0)),
                  pl.BlockSpec((mb, 1), lambda i: (i, 0))],
        out_specs=pl.BlockSpec((mb, d), lambda i: (i, 0)),
        out_shape=jax.ShapeDtypeStruct((m, d), f32))(x, dinv)


def _t2(ppix, bd, pgn):
    def kern(x_r, bd_r, pg_r, o_r):
        td = jnp.dot(x_r[...], bd_r[...], preferred_element_type=f32)
        oh = (pg_r[...] == lax.broadcasted_iota(i32, (B, N_PPI), 1)).astype(f32)
        o_r[...] = jnp.dot(oh, td, preferred_element_type=f32)

    return pl.pallas_call(
        kern, out_shape=jax.ShapeDtypeStruct((B, 128), f32))(ppix, bd, pgn)


def _add_gather_scale(a, t2, batch, dinv, mb=2000):
    m, d = a.shape

    def kern(a_r, t_r, bt_r, d_r, o_r):
        oh = (bt_r[...] == lax.broadcasted_iota(i32, (mb, B), 1)).astype(f32)
        q = jnp.dot(oh, t_r[...], preferred_element_type=f32)
        o_r[...] = (a_r[...] + q) * d_r[...]

    return pl.pallas_call(
        kern, grid=(m // mb,),
        in_specs=[pl.BlockSpec((mb, d), lambda i: (i, 0)),
                  pl.BlockSpec((B, d), lambda i: (0, 0)),
                  pl.BlockSpec((mb, 1), lambda i: (i, 0)),
                  pl.BlockSpec((mb, 1), lambda i: (i, 0))],
        out_specs=pl.BlockSpec((mb, d), lambda i: (i, 0)),
        out_shape=jax.ShapeDtypeStruct((m, d), f32))(a, t2, batch, dinv)


def _poolhead(x, batch, w1, b1, w2, b2):
    m, d = x.shape
    do1, do2 = w1.shape[1], w2.shape[1]

    def kern(x_r, bt_r, w1_r, b1_r, w2_r, b2_r, o_r):
        oh = (lax.broadcasted_iota(i32, (B, m), 0) == bt_r[...].T).astype(f32)
        ssum = jnp.dot(oh, x_r[...], preferred_element_type=f32)
        cnt = jnp.maximum(jnp.sum(oh, axis=1), 1.0)[:, None]
        pooled = ssum / cnt
        h = jnp.maximum(jnp.dot(pooled, w1_r[...], preferred_element_type=f32)
                        + b1_r[...], 0.0)
        o_r[...] = jnp.dot(h, w2_r[...], preferred_element_type=f32) + b2_r[...]

    return pl.pallas_call(
        kern, out_shape=jax.ShapeDtypeStruct((B, do2), f32))(
            x, batch, w1, b1.reshape(1, do1), w2, b2.reshape(1, do2))


def _final(x, batch, xmol, seq, pw1, pb1, pw2, pb2, fw1, fb1, fw2, fb2,
           ow, ob):
    m, d = x.shape

    def kern(x_r, bt_r, xm_r, sq_r, pw1_r, pb1_r, pw2_r, pb2_r, fw1_r, fb1_r,
             fw2_r, fb2_r, ow_r, ob_r, o_r):
        oh = (lax.broadcasted_iota(i32, (B, m), 0) == bt_r[...].T).astype(f32)
        ssum = jnp.dot(oh, x_r[...], preferred_element_type=f32)
        cnt = jnp.maximum(jnp.sum(oh, axis=1), 1.0)[:, None]
        pooled = ssum / cnt
        h = jnp.maximum(jnp.dot(pooled, pw1_r[...], preferred_element_type=f32)
                        + pb1_r[...], 0.0)
        px = jnp.dot(h, pw2_r[...], preferred_element_type=f32) + pb2_r[...]
        ohs = (sq_r[...] == lax.broadcasted_iota(i32, (B, B), 1)).astype(f32)
        psel = jnp.dot(ohs, px, preferred_element_type=f32)
        hc = jnp.maximum(
            jnp.dot(xm_r[...], fw1_r[0], preferred_element_type=f32) +
            jnp.dot(psel, fw1_r[1], preferred_element_type=f32) +
            fb1_r[...], 0.0)
        h2 = jnp.maximum(jnp.dot(hc, fw2_r[...], preferred_element_type=f32) +
                         fb2_r[...], 0.0)
        o_r[...] = jnp.dot(h2, ow_r[...], preferred_element_type=f32) + ob_r[...]

    return pl.pallas_call(
        kern, out_shape=jax.ShapeDtypeStruct((B, 1), f32))(
            x, batch, xmol, seq, pw1, pb1.reshape(1, 1024), pw2,
            pb2.reshape(1, 128), fw1.reshape(2, 128, 1024), fb1.reshape(1, 1024),
            fw2, fb2.reshape(1, 512), ow, ob.reshape(1, 1))


def kernel(mol_x, mol_edge_index, mol_batch, seq_num, ppi_edge, ppi_features,
           pro_x, pro_edge_index, pro_graph_num, pro_batch, params):
    p = params
    wmg1, bmg1 = p['mg1']; wmg2, bmg2 = p['mg2']; wmg3, bmg3 = p['mg3']
    wpg1, bpg1 = p['pg1']; wpg2, bpg2 = p['pg2']; wpg3, bpg3 = p['pg3']
    wppi1, bppi1 = p['ppig1']; wppi2, bppi2 = p['ppig2']

    wa = wpg2[:64] + wpg2[64:]
    wb = wpg2[:64] - wpg2[64:]

    m_src, m_dst = mol_edge_index[0], mol_edge_index[1]
    i_src, i_dst = ppi_edge[0], ppi_edge[1]
    r_src, r_dst = pro_edge_index[0], pro_edge_index[1]
    pgn = pro_graph_num.astype(i32).reshape(B, 1)
    seq = seq_num.astype(i32).reshape(B, 1)
    mbat = mol_batch.astype(i32).reshape(N_MOL, 1)
    pbat = pro_batch.astype(i32).reshape(N_PRO, 1)

    def seg(vals, dst, n):
        return jax.ops.segment_sum(vals, dst, num_segments=n)

    dinv_m = _dinv(seg(jnp.ones((E_MOL,), f32), m_dst, N_MOL))
    dinv_i = _dinv(seg(jnp.ones((E_PPI,), f32), i_dst, N_PPI))
    dinv_r = _dinv(seg(jnp.ones((E_PRO,), f32), r_dst, N_PRO))

    g1m = _scale(mol_x, dinv_m)
    s1m = seg(g1m[m_src], m_dst, N_MOL)
    g2m = _combine_mm(s1m, g1m, dinv_m, wmg1, bpost=bmg1, relu_post=True,
                      rowscale=dinv_m)
    s2m = seg(g2m[m_src], m_dst, N_MOL)
    x2m = _combine_mm(s2m, g2m, dinv_m, wmg2, bpost=bmg2, relu_post=True)
    g3m = _mm(x2m, wmg3, rowscale=dinv_m)
    s3m = seg(g3m[m_src], m_dst, N_MOL)
    x3m = _combine(s3m, g3m, dinv_m, bpre=bmg3, relu=True)
    xmol = _poolhead(x3m, mbat, p['mfc1'][0], p['mfc1'][1],
                     p['mfc2'][0], p['mfc2'][1])

    g1i = _mm(ppi_features, wppi1, rowscale=dinv_i)
    s1i = seg(g1i[i_src], i_dst, N_PPI)
    g2i = _combine_mm(s1i, g1i, dinv_i, wppi2, bpre=bppi1, relu_pre=True,
                      rowscale=dinv_i)
    s2i = seg(g2i[i_src], i_dst, N_PPI)
    hpp = _combine_mm(s2i, g2i, dinv_i, p['ppifc1'][0], bpre=bppi2,
                      relu_pre=True, bpost=p['ppifc1'][1], relu_post=True)
    ppix = _mm(hpp, p['ppifc2'][0], b=p['ppifc2'][1])
    t2tab = _t2(ppix, wb, pgn)

    g1r = _mm(pro_x, wpg1, rowscale=dinv_r)
    s1r = seg(g1r[r_src], r_dst, N_PRO)
    pd = _combine_mm(s1r, g1r, dinv_r, wa, bpre=bpg1, relu_pre=True)
    gpro2 = _add_gather_scale(pd, t2tab, pbat, dinv_r)
    s2r = seg(gpro2[r_src], r_dst, N_PRO)
    g3r = _combine_mm(s2r, gpro2, dinv_r, wpg3, bpre=bpg2, relu_pre=True,
                      rowscale=dinv_r)
    s3r = seg(g3r[r_src], r_dst, N_PRO)
    x3r = _combine(s3r, g3r, dinv_r, bpre=bpg3, relu=True)

    return _final(x3r, pbat, xmol, seq, p['pfc1'][0], p['pfc1'][1],
                  p['pfc2'][0], p['pfc2'][1], p['fc1'][0], p['fc1'][1],
                  p['fc2'][0], p['fc2'][1], p['out'][0], p['out'][1])
